# baseline (speedup 1.0000x reference)
import jax
import jax.numpy as jnp
from jax.experimental import pallas as pl

N = 100000
E = 1600000
H = 64
NUM_LAYERS = 8
C = 3
G = 64


def _sage_conv(x, src, dst, Wl, bl, Wr):
    msg = x[src]
    agg = jax.ops.segment_sum(msg, dst, num_segments=N)
    deg = jax.ops.segment_sum(jnp.ones((src.shape[0],), x.dtype), dst, num_segments=N)
    mean = agg / jnp.clip(deg, 1.0)[:, None]
    return mean @ Wl + bl + x @ Wr


def kernel(x, edge_index, batch, W1l, b1l, W1r, Wls, bls, Wrs, Wlin1, blin1, Wlin2, blin2):
    src = edge_index[0]
    dst = edge_index[1]
    dst, src = jax.lax.sort((dst, src), num_keys=1)
    starts = jnp.searchsorted(dst, jnp.arange(0, N + 1, 625, dtype=jnp.int32))
    src = src + (starts[0] - starts[0]).astype(jnp.int32)
    h = jax.nn.relu(_sage_conv(x, src, dst, W1l, b1l, W1r))
    for i in range(NUM_LAYERS - 1):
        h = jax.nn.relu(_sage_conv(h, src, dst, Wls[i], bls[i], Wrs[i]))
    g = jax.ops.segment_sum(h, batch, num_segments=G)
    g = jax.nn.relu(g @ Wlin1 + blin1)
    out = g @ Wlin2 + blin2
    return jax.nn.log_softmax(out, axis=-1)


# speedup vs baseline: 5.5268x; 5.5268x over previous
import functools

import jax
import jax.numpy as jnp
from jax import lax
from jax.experimental import pallas as pl
from jax.experimental.pallas import tpu as pltpu
from jax.experimental.pallas import tpu_sc as plsc

N = 100000
E = 1600000
H = 64
HP = 128
NUM_LAYERS = 8
C = 3
G = 64

NC = 2
NS = 16
CH = 6272
NCHUNK = 16
SLOP = 128
CHP = CH + SLOP
NROW = NCHUNK * CH
BLK_E = 512
BODY_E = 1024
EPAD = E + 1024

BLK = 512
GRID = NROW // BLK

ZT = CHP // NS
CT = CH // NS


def _sc_spmm(h, srcs2, dsts2, starts16, zerosP):
    mesh = plsc.VectorSubcoreMesh(core_axis_name="c", subcore_axis_name="s")

    @functools.partial(
        pl.kernel,
        mesh=mesh,
        out_type=jax.ShapeDtypeStruct((NROW, HP), jnp.float32),
        scratch_types=[
            pltpu.VMEM_SHARED((CHP, HP), jnp.float32),
            pltpu.VMEM((8, 128), jnp.int32),
            pltpu.VMEM((8, 128), jnp.int32),
            pltpu.VMEM((8, 128), jnp.int32),
            pltpu.VMEM((BLK_E, HP), jnp.float32),
            pltpu.VMEM((32,), jnp.int32),
            pltpu.SemaphoreType.DMA,
        ],
    )
    def k(h_hbm, src_hbm, dst_hbm, st_hbm,
          z_hbm, out_hbm, acc, srcv, dstv, dlocv, rows, stv_v, sem):
        cid = lax.axis_index("c")
        sid = lax.axis_index("s")
        lanes = lax.iota(jnp.int32, 16)
        pltpu.sync_copy(st_hbm, stv_v)
        stv_r0 = stv_v[pl.ds(0, 16)]
        stv_r1 = stv_v[pl.ds(16, 16)]
        stv = ([stv_r0[j] for j in range(16)]
               + [stv_r1[j] for j in range(NCHUNK + 1 - 16)])

        for cc in range(NCHUNK // NC):
            ci = cid * (NCHUNK // NC) + cc
            base_c = ci * CH

            pltpu.sync_copy(z_hbm.at[pl.ds(sid * ZT, ZT), :],
                            acc.at[pl.ds(sid * ZT, ZT), :])
            plsc.subcore_barrier()

            nh = NCHUNK // NC
            s = jnp.where(cid == 0, stv[cc], stv[nh + cc])
            t = jnp.where(cid == 0, stv[cc + 1], stv[nh + cc + 1])
            ln = t - s
            li = s + lax.shift_right_logical(ln * sid, 4)
            hi = s + lax.shift_right_logical(ln * (sid + 1), 4)
            a0 = li - lax.bitwise_and(li, BODY_E - 1)
            nb = lax.shift_right_logical(hi - a0 + BODY_E - 1, 10)

            def body(kb, _):
                e0 = a0 + kb * BODY_E
                r0 = pl.multiple_of(lax.shift_right_logical(e0, 7), 8)
                pltpu.sync_copy(src_hbm.at[pl.ds(r0, 8), :], srcv)
                pltpu.sync_copy(dst_hbm.at[pl.ds(r0, 8), :], dstv)
                lo_v = li - e0
                hi_v = hi - e0
                for half in range(2):
                    hs = [
                        pltpu.async_copy(
                            h_hbm.at[srcv.at[half * 4 + j]],
                            rows.at[pl.ds(j * 128, 128), :], sem)
                        for j in range(4)
                    ]
                    for jr in range(4):
                        for jc in range(8):
                            gofs = half * 512 + jr * 128 + jc * 16
                            lane_id = lanes + gofs
                            dv = dstv[half * 4 + jr, pl.ds(jc * 16, 16)]
                            valid = (lane_id >= lo_v) & (lane_id < hi_v)
                            dlocv[half * 4 + jr, pl.ds(jc * 16, 16)] = (
                                jnp.where(valid, dv - base_c, CH + lanes))
                    for hcp in hs:
                        hcp.wait()
                    for j in range(4):
                        pltpu.sync_copy(
                            rows.at[pl.ds(j * 128, 128), :],
                            acc.at[dlocv.at[half * 4 + j]], add=True)
                return _

            lax.fori_loop(0, nb, body, None)
            plsc.subcore_barrier()

            pltpu.sync_copy(
                acc.at[pl.ds(sid * CT, CT), :],
                out_hbm.at[pl.ds(base_c + sid * CT, CT), :],
            )
            plsc.subcore_barrier()

    return k(h, srcs2, dsts2, starts16, zerosP)


def _tc_layer1(agg0, h0, W1lp, b1p, W1rp):
    def body(a_ref, x_ref, wl_ref, b_ref, wr_ref, h_ref, inv_ref):
        a = a_ref[...]
        deg = a[:, 1:2]
        invd = 1.0 / jnp.maximum(deg, 1.0)
        mean1 = a[:, 0:1] * invd
        xcol = x_ref[:, 0:1]
        h_ref[...] = jnp.maximum(
            mean1 * wl_ref[...] + xcol * wr_ref[...] + b_ref[...], 0.0)
        inv_ref[...] = invd

    return pl.pallas_call(
        body,
        grid=(GRID,),
        in_specs=[
            pl.BlockSpec((BLK, HP), lambda i: (i, 0)),
            pl.BlockSpec((BLK, HP), lambda i: (i, 0)),
            pl.BlockSpec((1, HP), lambda i: (0, 0)),
            pl.BlockSpec((1, HP), lambda i: (0, 0)),
            pl.BlockSpec((1, HP), lambda i: (0, 0)),
        ],
        out_specs=[
            pl.BlockSpec((BLK, HP), lambda i: (i, 0)),
            pl.BlockSpec((BLK, 1), lambda i: (i, 0)),
        ],
        out_shape=[
            jax.ShapeDtypeStruct((NROW, HP), jnp.float32),
            jax.ShapeDtypeStruct((NROW, 1), jnp.float32),
        ],
    )(agg0, h0, W1lp, b1p, W1rp)


def _tc_layer(agg, hcur, invd, Wlp, bp, Wrp):
    def body(a_ref, h_ref, inv_ref, wl_ref, b_ref, wr_ref, o_ref):
        mean = a_ref[...] * inv_ref[...]
        o = (jnp.dot(mean, wl_ref[...], preferred_element_type=jnp.float32)
             + jnp.dot(h_ref[...], wr_ref[...], preferred_element_type=jnp.float32)
             + b_ref[...])
        o_ref[...] = jnp.maximum(o, 0.0)

    return pl.pallas_call(
        body,
        grid=(GRID,),
        in_specs=[
            pl.BlockSpec((BLK, HP), lambda i: (i, 0)),
            pl.BlockSpec((BLK, HP), lambda i: (i, 0)),
            pl.BlockSpec((BLK, 1), lambda i: (i, 0)),
            pl.BlockSpec((HP, HP), lambda i: (0, 0)),
            pl.BlockSpec((1, HP), lambda i: (0, 0)),
            pl.BlockSpec((HP, HP), lambda i: (0, 0)),
        ],
        out_specs=pl.BlockSpec((BLK, HP), lambda i: (i, 0)),
        out_shape=jax.ShapeDtypeStruct((NROW, HP), jnp.float32),
    )(agg, hcur, invd, Wlp, bp, Wrp)


def _tc_head(h8, batchp, Wlin1p, blin1p, Wlin2p, blin2p):
    def body(h_ref, b_ref, wl1_ref, bl1_ref, wl2_ref, bl2_ref, o_ref, acc_ref):
        i = pl.program_id(0)

        @pl.when(i == 0)
        def _():
            acc_ref[...] = jnp.zeros((G, HP), jnp.float32)

        grp = lax.broadcasted_iota(jnp.int32, (1, G), 1)
        onehot = (b_ref[...] == grp).astype(jnp.float32)
        acc_ref[...] += lax.dot_general(
            onehot, h_ref[...], (((0,), (0,)), ((), ())),
            preferred_element_type=jnp.float32)

        @pl.when(i == GRID - 1)
        def _():
            gg = acc_ref[...]
            t = jnp.maximum(
                jnp.dot(gg, wl1_ref[...], preferred_element_type=jnp.float32)
                + bl1_ref[...], 0.0)
            o = (jnp.dot(t, wl2_ref[...], preferred_element_type=jnp.float32)
                 + bl2_ref[...])
            m = jnp.max(o, axis=-1, keepdims=True)
            lse = jnp.log(jnp.sum(jnp.exp(o - m), axis=-1, keepdims=True)) + m
            o_ref[...] = o - lse

    return pl.pallas_call(
        body,
        grid=(GRID,),
        in_specs=[
            pl.BlockSpec((BLK, HP), lambda i: (i, 0)),
            pl.BlockSpec((BLK, 1), lambda i: (i, 0)),
            pl.BlockSpec((HP, HP), lambda i: (0, 0)),
            pl.BlockSpec((1, HP), lambda i: (0, 0)),
            pl.BlockSpec((HP, HP), lambda i: (0, 0)),
            pl.BlockSpec((1, HP), lambda i: (0, 0)),
        ],
        out_specs=pl.BlockSpec((G, HP), lambda i: (0, 0)),
        out_shape=jax.ShapeDtypeStruct((G, HP), jnp.float32),
        scratch_shapes=[pltpu.VMEM((G, HP), jnp.float32)],
    )(h8, batchp, Wlin1p, blin1p, Wlin2p, blin2p)


def kernel(x, edge_index, batch, W1l, b1l, W1r, Wls, bls, Wrs,
           Wlin1, blin1, Wlin2, blin2):
    f32 = jnp.float32
    i32 = jnp.int32
    src = edge_index[0].astype(i32)
    dst = edge_index[1].astype(i32)

    dsts, srcs = lax.sort((dst, src), num_keys=1)
    bounds = (jnp.arange(1, NCHUNK, dtype=i32) * CH).astype(i32)
    s17 = jnp.searchsorted(dsts, bounds).astype(i32)
    starts16 = jnp.concatenate(
        [jnp.zeros((1,), i32), s17,
         jnp.full((32 - NCHUNK,), E, i32)])

    npad = EPAD - E
    srcs2 = jnp.concatenate(
        [srcs, (jnp.arange(npad, dtype=i32) * 131) % N]).reshape(EPAD // 128, 128)
    dsts2 = jnp.concatenate(
        [dsts, jnp.zeros((npad,), i32)]).reshape(EPAD // 128, 128)

    h0 = jnp.zeros((NROW, HP), f32)
    h0 = h0.at[:N, 0].set(x[:, 0])
    h0 = h0.at[:N, 1].set(1.0)
    zerosP = jnp.zeros((CHP, HP), f32)

    W1lp = jnp.zeros((1, HP), f32).at[:, :H].set(W1l)
    W1rp = jnp.zeros((1, HP), f32).at[:, :H].set(W1r)
    b1p = jnp.zeros((1, HP), f32).at[:, :H].set(b1l.reshape(1, H))
    Wlsp = jnp.zeros((NUM_LAYERS - 1, HP, HP), f32).at[:, :H, :H].set(Wls)
    Wrsp = jnp.zeros((NUM_LAYERS - 1, HP, HP), f32).at[:, :H, :H].set(Wrs)
    blsp = jnp.zeros((NUM_LAYERS - 1, 1, HP), f32).at[:, 0, :H].set(bls)

    agg0 = _sc_spmm(h0, srcs2, dsts2, starts16, zerosP)
    h, invd = _tc_layer1(agg0, h0, W1lp, b1p, W1rp)

    for i in range(NUM_LAYERS - 1):
        agg = _sc_spmm(h, srcs2, dsts2, starts16, zerosP)
        h = _tc_layer(agg, h, invd, Wlsp[i], blsp[i], Wrsp[i])

    batchp = jnp.concatenate(
        [batch.astype(i32), jnp.full((NROW - N,), G, i32)]).reshape(NROW, 1)
    Wlin1p = jnp.zeros((HP, HP), f32).at[:H, :H].set(Wlin1)
    blin1p = jnp.zeros((1, HP), f32).at[:, :H].set(blin1.reshape(1, H))
    Wlin2p = jnp.zeros((HP, HP), f32).at[:H, :C].set(Wlin2)
    blin2p = jnp.full((1, HP), -1e30, f32).at[0, :C].set(blin2)
    out = _tc_head(h, batchp, Wlin1p, blin1p, Wlin2p, blin2p)
    return out[:, :C]
